# baseline (speedup 1.0000x reference)
import functools

import jax
import jax.numpy as jnp
from jax import lax
from jax.experimental import pallas as pl
from jax.experimental.pallas import tpu as pltpu
from jax.experimental.pallas import tpu_sc as plsc

R = 2
N = 10000
NT = R * N
E = 160000
D = 128
OUT = 112
G = 128
NUM_LAYERS = 4

NS = 16
NCORES = 2

CHUNK = 88
SBS = 4
PAIRS = 29
EPT = PAIRS * SBS * CHUNK
EPAD = EPT - E // NS

ACC_ROWS = 10112
TPT = 632
NTRASH = ACC_ROWS - N
XT = 20160
CANVAS = N + ACC_ROWS


@functools.cache
def _sc_kernels():
    mesh = plsc.VectorSubcoreMesh(
        core_axis_name="c", subcore_axis_name="s", num_cores=NCORES, num_subcores=NS
    )

    @functools.partial(
        pl.kernel,
        out_type=jax.ShapeDtypeStruct((NCORES * ACC_ROWS, D), jnp.float32),
        mesh=mesh,
        scratch_types=[
            pltpu.VMEM((2, SBS, 2, CHUNK), jnp.int32),
            pltpu.VMEM((SBS, CHUNK, D), jnp.float32),
            pltpu.VMEM_SHARED((ACC_ROWS, D), jnp.float32),
        ]
        + [pltpu.SemaphoreType.DMA] * (2 * SBS),
    )
    def mp_kernel(xs_hbm, idx_hbm, z_hbm, out_hbm, idxb, gbuf, acc, *sems):
        c = lax.axis_index("c")
        s = lax.axis_index("s")
        semg = sems[:SBS]
        semsc = sems[SBS:]

        def idxload(k, buf):
            pltpu.sync_copy(idx_hbm.at[c, s, k], idxb.at[buf])

        def gather_start(i, buf):
            return pltpu.async_copy(
                xs_hbm.at[idxb.at[buf, i, 0]], gbuf.at[i], semg[i])

        def scatter_start(i, buf):
            return pltpu.async_copy(
                gbuf.at[i], acc.at[idxb.at[buf, i, 1]], semsc[i], add=True)

        def gather_wait(i, buf):
            pltpu.make_async_copy(
                xs_hbm.at[idxb.at[buf, i, 0]], gbuf.at[i], semg[i]).wait()

        idxload(0, 0)
        for i in range(SBS):
            gather_start(i, 0)
        base = pl.multiple_of(s * TPT, 8)
        pltpu.sync_copy(z_hbm.at[pl.ds(base, TPT)], acc.at[pl.ds(base, TPT)])
        plsc.subcore_barrier()

        def body(k, carry):
            p = k & 1
            scs = []
            for i in range(SBS):
                gather_wait(i, p)
                scs.append(scatter_start(i, p))

            @pl.when(k < PAIRS - 1)
            def _():
                idxload(k + 1, 1 - p)
                for i in range(SBS):
                    scs[i].wait()
                    gather_start(i, 1 - p)

            @pl.when(k == PAIRS - 1)
            def _():
                for i in range(SBS):
                    scs[i].wait()

            return carry

        lax.fori_loop(0, PAIRS, body, 0)
        plsc.subcore_barrier()
        obase = pl.multiple_of(c * ACC_ROWS + base, 8)
        pltpu.sync_copy(acc.at[pl.ds(base, TPT)], out_hbm.at[pl.ds(obase, TPT)])

    @functools.partial(
        pl.kernel,
        out_type=jax.ShapeDtypeStruct((NCORES * ACC_ROWS, D), jnp.float32),
        mesh=mesh,
        scratch_types=[
            pltpu.VMEM((2, SBS, 2, CHUNK), jnp.int32),
            pltpu.VMEM((CHUNK, D), jnp.float32),
            pltpu.VMEM_SHARED((ACC_ROWS, D), jnp.float32),
        ]
        + [pltpu.SemaphoreType.DMA] * SBS,
    )
    def deg_kernel(idx_hbm, z_hbm, ones_hbm, out_hbm, idxb, ones_v, acc, *sems):
        c = lax.axis_index("c")
        s = lax.axis_index("s")

        def idxload(k, buf):
            pltpu.sync_copy(idx_hbm.at[c, s, k], idxb.at[buf])

        idxload(0, 0)
        pltpu.sync_copy(ones_hbm, ones_v)
        base = pl.multiple_of(s * TPT, 8)
        pltpu.sync_copy(z_hbm.at[pl.ds(base, TPT)], acc.at[pl.ds(base, TPT)])
        plsc.subcore_barrier()

        def body(k, carry):
            p = k & 1
            scs = [
                pltpu.async_copy(ones_v, acc.at[idxb.at[p, i, 1]], sems[i], add=True)
                for i in range(SBS)
            ]

            @pl.when(k < PAIRS - 1)
            def _():
                idxload(k + 1, 1 - p)

            for i in range(SBS):
                scs[i].wait()
            return carry

        lax.fori_loop(0, PAIRS, body, 0)
        plsc.subcore_barrier()
        obase = pl.multiple_of(c * ACC_ROWS + base, 8)
        pltpu.sync_copy(acc.at[pl.ds(base, TPT)], out_hbm.at[pl.ds(obase, TPT)])

    return mp_kernel, deg_kernel


def _bn(h, g, b):
    mu = jnp.mean(h, axis=0, keepdims=True)
    va = jnp.mean((h - mu) * (h - mu), axis=0, keepdims=True)
    return (h - mu) / jnp.sqrt(va + 1e-5) * g + b


def _dot(a, b):
    return jnp.dot(a, b, preferred_element_type=jnp.float32)


def _conv_stage(xf, dinvb, W1t, b1, g1, be1, W2t, b2, Wgt):
    h = _dot(xf, W1t) + b1
    h = jnp.maximum(_bn(h, g1, be1), 0.0)
    h = _dot(h, W2t) + b2
    xt = _dot(h, Wgt)
    return xt * dinvb


def _write_xs(xs_out, xs):
    xs_out[0:NT, :] = xs
    xs_out[NT:XT, :] = jnp.zeros((XT - NT, D), jnp.float32)


def _pool_y(P, m, fWt, fb):
    pooled = lax.dot_general(P, m, (((0,), (0,)), ((), ())), preferred_element_type=jnp.float32)
    return _dot(pooled, fWt) + fb


def _tc0_body(xf_ref, xm_ref, dinvb_ref, P_ref, W1t, b1, g1, be1, W2t, b2, Wgt, fWt, fb,
              xs_out, y_out):
    xs = _conv_stage(xf_ref[...], dinvb_ref[...], W1t[...], b1[...], g1[...], be1[...],
                     W2t[...], b2[...], Wgt[...])
    _write_xs(xs_out, xs)
    y_out[...] = _pool_y(P_ref[...], xm_ref[...], fWt[...], fb[...])


def _consume(agg_ref, xs_ref, dinvb, bgb, bng, bnb):
    ht = (agg_ref[0:NT, :] + xs_ref[0:NT, :]) * dinvb + bgb
    return jnp.maximum(_bn(ht, bng, bnb), 0.0)


def _tc_mid_body(agg_ref, xsp_ref, dinvb_ref, P_ref, bgb, bng, bnb, W1t, b1, g1, be1,
                 W2t, b2, Wgt, fWt, fb, xs_out, y_out):
    dinvb = dinvb_ref[...]
    hn = _consume(agg_ref, xsp_ref, dinvb, bgb[...], bng[...], bnb[...])
    m = 0.5 * (hn[0:N, :] + hn[N:NT, :])
    y_out[...] = _pool_y(P_ref[...], m, fWt[...], fb[...])
    xs = _conv_stage(hn, dinvb, W1t[...], b1[...], g1[...], be1[...], W2t[...], b2[...],
                     Wgt[...])
    _write_xs(xs_out, xs)


def _tc_final_body(agg_ref, xsp_ref, dinvb_ref, P_ref, bgb, bng, bnb, fWt, fb, y_out):
    hn = _consume(agg_ref, xsp_ref, dinvb_ref[...], bgb[...], bng[...], bnb[...])
    m = 0.5 * (hn[0:N, :] + hn[N:NT, :])
    y_out[...] = _pool_y(P_ref[...], m, fWt[...], fb[...])


_XS_TYPE = jax.ShapeDtypeStruct((XT, D), jnp.float32)
_Y_TYPE = jax.ShapeDtypeStruct((G, OUT), jnp.float32)

_tc0 = pl.pallas_call(_tc0_body, out_shape=[_XS_TYPE, _Y_TYPE])
_tc_mid = pl.pallas_call(_tc_mid_body, out_shape=[_XS_TYPE, _Y_TYPE])
_tc_final = pl.pallas_call(_tc_final_body, out_shape=_Y_TYPE)


def kernel(x, edge_index, batch, params):
    f32 = jnp.float32
    drop = jax.random.bernoulli(jax.random.key(42), 0.2, (R, N))
    scale = 1.0 - drop.astype(f32)
    xf0 = jnp.concatenate([x * scale[0][:, None], x * scale[1][:, None]], axis=0)
    xm0 = x * ((scale[0] + scale[1]) * 0.5)[:, None]

    em = jnp.max(edge_index) + 1
    row0 = edge_index[0].reshape(NS, E // NS)
    col0 = edge_index[1].reshape(NS, E // NS)
    padr = jnp.zeros((NS, EPAD), jnp.int32)
    padc = N + (jnp.broadcast_to(jnp.arange(EPAD, dtype=jnp.int32) % NTRASH, (NS, EPAD)))
    colp = jnp.concatenate([col0, padc], axis=1)
    rows0 = jnp.concatenate([row0, padr], axis=1)
    rows_c = jnp.stack([rows0, rows0 + em])
    cols_c = jnp.broadcast_to(colp[None], (NCORES, NS, EPT))
    idx_mp = jnp.stack(
        [rows_c.reshape(NCORES, NS, PAIRS, SBS, CHUNK),
         cols_c.reshape(NCORES, NS, PAIRS, SBS, CHUNK)], axis=4)
    zinit = jnp.zeros((ACC_ROWS, D), f32)

    mp_kernel, deg_kernel = _sc_kernels()

    ztrash = jnp.zeros((NTRASH, D), f32)
    ztail = jnp.zeros((CANVAS - ACC_ROWS, D), f32)

    def recombine(out_cat):
        out1 = jnp.concatenate([out_cat[ACC_ROWS : ACC_ROWS + N], ztrash], axis=0)
        canvas = jnp.concatenate([out_cat[0:ACC_ROWS], ztail], axis=0)
        return lax.dynamic_update_slice(canvas, out1, (em, 0))

    deg_cat = deg_kernel(idx_mp, zinit, jnp.ones((CHUNK, D), f32))
    deg = recombine(deg_cat)[0:NT, 0] + 1.0
    dinv = jnp.where(deg > 0, lax.rsqrt(deg), 0.0)
    dinvb = jnp.broadcast_to(dinv[:, None], (NT, D))

    P = (batch[:, None] == jnp.arange(G, dtype=batch.dtype)[None, :]).astype(f32)

    def r2(v):
        return v.reshape(1, -1)

    convs = params["convs"]
    fcs = params["fcs"]
    bns = params["bns"]

    c0 = convs[0]
    xs, y = _tc0(
        xf0, xm0, dinvb, P,
        c0["W1"].T, r2(c0["b1"]), r2(c0["g1"]), r2(c0["be1"]),
        c0["W2"].T, r2(c0["b2"]), c0["Wg"].T,
        fcs[0]["W"].T, r2(fcs[0]["b"]),
    )
    out = y
    for i in range(NUM_LAYERS):
        agg = recombine(mp_kernel(xs, idx_mp, zinit))
        bgb = r2(convs[i]["bg"])
        bng, bnb = r2(bns[i]["g"]), r2(bns[i]["b"])
        fWt, fb = fcs[i + 1]["W"].T, r2(fcs[i + 1]["b"])
        if i < NUM_LAYERS - 1:
            cn = convs[i + 1]
            xs_next, y = _tc_mid(
                agg, xs, dinvb, P, bgb, bng, bnb,
                cn["W1"].T, r2(cn["b1"]), r2(cn["g1"]), r2(cn["be1"]),
                cn["W2"].T, r2(cn["b2"]), cn["Wg"].T,
                fWt, fb,
            )
            xs = xs_next
        else:
            y = _tc_final(agg, xs, dinvb, P, bgb, bng, bnb, fWt, fb)
        out = out + y
    return out


# speedup vs baseline: 1.2688x; 1.2688x over previous
import functools

import jax
import jax.numpy as jnp
from jax import lax
from jax.experimental import pallas as pl
from jax.experimental.pallas import tpu as pltpu
from jax.experimental.pallas import tpu_sc as plsc

R = 2
N = 10000
NT = R * N
E = 160000
D = 128
OUT = 112
G = 128
NUM_LAYERS = 4

NS = 16
NCORES = 2

CHUNK = 120
SBS = 3
PAIRS = 28
EPT = PAIRS * SBS * CHUNK
EPAD = EPT - E // NS

ACC_ROWS = 10112
TPT = 632
NTRASH = ACC_ROWS - N
XT = 20160
CANVAS = N + ACC_ROWS


@functools.cache
def _sc_kernels():
    mesh = plsc.VectorSubcoreMesh(
        core_axis_name="c", subcore_axis_name="s", num_cores=NCORES, num_subcores=NS
    )

    @functools.partial(
        pl.kernel,
        out_type=jax.ShapeDtypeStruct((NCORES * ACC_ROWS, D), jnp.float32),
        mesh=mesh,
        scratch_types=[
            pltpu.VMEM((2, SBS, 2, CHUNK), jnp.int32),
            pltpu.VMEM((SBS, CHUNK, D), jnp.float32),
            pltpu.VMEM_SHARED((ACC_ROWS, D), jnp.float32),
        ]
        + [pltpu.SemaphoreType.DMA] * (2 * SBS),
    )
    def mp_kernel(xs_hbm, idx_hbm, z_hbm, out_hbm, idxb, gbuf, acc, *sems):
        c = lax.axis_index("c")
        s = lax.axis_index("s")
        semg = sems[:SBS]
        semsc = sems[SBS:]

        def idxload(k, buf):
            pltpu.sync_copy(idx_hbm.at[c, s, k], idxb.at[buf])

        def gather_start(i, buf):
            return pltpu.async_copy(
                xs_hbm.at[idxb.at[buf, i, 0]], gbuf.at[i], semg[i])

        def scatter_start(i, buf):
            return pltpu.async_copy(
                gbuf.at[i], acc.at[idxb.at[buf, i, 1]], semsc[i], add=True)

        def gather_wait(i, buf):
            pltpu.make_async_copy(
                xs_hbm.at[idxb.at[buf, i, 0]], gbuf.at[i], semg[i]).wait()

        idxload(0, 0)
        for i in range(SBS):
            gather_start(i, 0)
        base = pl.multiple_of(s * TPT, 8)
        pltpu.sync_copy(z_hbm.at[pl.ds(base, TPT)], acc.at[pl.ds(base, TPT)])
        plsc.subcore_barrier()

        def body(k, carry):
            p = k & 1
            scs = []
            for i in range(SBS):
                gather_wait(i, p)
                scs.append(scatter_start(i, p))

            @pl.when(k < PAIRS - 1)
            def _():
                idxload(k + 1, 1 - p)
                for i in range(SBS):
                    scs[i].wait()
                    gather_start(i, 1 - p)

            @pl.when(k == PAIRS - 1)
            def _():
                for i in range(SBS):
                    scs[i].wait()

            return carry

        lax.fori_loop(0, PAIRS, body, 0)
        plsc.subcore_barrier()
        obase = pl.multiple_of(c * ACC_ROWS + base, 8)
        pltpu.sync_copy(acc.at[pl.ds(base, TPT)], out_hbm.at[pl.ds(obase, TPT)])

    @functools.partial(
        pl.kernel,
        out_type=jax.ShapeDtypeStruct((NCORES * ACC_ROWS, D), jnp.float32),
        mesh=mesh,
        scratch_types=[
            pltpu.VMEM((2, SBS, 2, CHUNK), jnp.int32),
            pltpu.VMEM((CHUNK, D), jnp.float32),
            pltpu.VMEM_SHARED((ACC_ROWS, D), jnp.float32),
        ]
        + [pltpu.SemaphoreType.DMA] * SBS,
    )
    def deg_kernel(idx_hbm, z_hbm, ones_hbm, out_hbm, idxb, ones_v, acc, *sems):
        c = lax.axis_index("c")
        s = lax.axis_index("s")

        def idxload(k, buf):
            pltpu.sync_copy(idx_hbm.at[c, s, k], idxb.at[buf])

        idxload(0, 0)
        pltpu.sync_copy(ones_hbm, ones_v)
        base = pl.multiple_of(s * TPT, 8)
        pltpu.sync_copy(z_hbm.at[pl.ds(base, TPT)], acc.at[pl.ds(base, TPT)])
        plsc.subcore_barrier()

        def body(k, carry):
            p = k & 1
            scs = [
                pltpu.async_copy(ones_v, acc.at[idxb.at[p, i, 1]], sems[i], add=True)
                for i in range(SBS)
            ]

            @pl.when(k < PAIRS - 1)
            def _():
                idxload(k + 1, 1 - p)

            for i in range(SBS):
                scs[i].wait()
            return carry

        lax.fori_loop(0, PAIRS, body, 0)
        plsc.subcore_barrier()
        obase = pl.multiple_of(c * ACC_ROWS + base, 8)
        pltpu.sync_copy(acc.at[pl.ds(base, TPT)], out_hbm.at[pl.ds(obase, TPT)])

    return mp_kernel, deg_kernel


def _bn(h, g, b):
    mu = jnp.mean(h, axis=0, keepdims=True)
    va = jnp.mean((h - mu) * (h - mu), axis=0, keepdims=True)
    return (h - mu) / jnp.sqrt(va + 1e-5) * g + b


def _dot(a, b):
    return jnp.dot(a, b, preferred_element_type=jnp.float32)


def _conv_stage(xf, dinvb, W1t, b1, g1, be1, W2t, b2, Wgt):
    h = _dot(xf, W1t) + b1
    h = jnp.maximum(_bn(h, g1, be1), 0.0)
    h = _dot(h, W2t) + b2
    xt = _dot(h, Wgt)
    return xt * dinvb


def _write_xs(xs_out, xs):
    xs_out[0:NT, :] = xs
    xs_out[NT:XT, :] = jnp.zeros((XT - NT, D), jnp.float32)


def _pool_y(P, m, fWt, fb):
    pooled = lax.dot_general(P, m, (((0,), (0,)), ((), ())), preferred_element_type=jnp.float32)
    return _dot(pooled, fWt) + fb


def _tc0_body(xf_ref, xm_ref, dinvb_ref, P_ref, W1t, b1, g1, be1, W2t, b2, Wgt, fWt, fb,
              xs_out, y_out):
    xs = _conv_stage(xf_ref[...], dinvb_ref[...], W1t[...], b1[...], g1[...], be1[...],
                     W2t[...], b2[...], Wgt[...])
    _write_xs(xs_out, xs)
    y_out[...] = _pool_y(P_ref[...], xm_ref[...], fWt[...], fb[...])


def _consume(agg_ref, xs_ref, dinvb, bgb, bng, bnb):
    ht = (agg_ref[0:NT, :] + xs_ref[0:NT, :]) * dinvb + bgb
    return jnp.maximum(_bn(ht, bng, bnb), 0.0)


def _tc_mid_body(agg_ref, xsp_ref, dinvb_ref, P_ref, bgb, bng, bnb, W1t, b1, g1, be1,
                 W2t, b2, Wgt, fWt, fb, xs_out, y_out):
    dinvb = dinvb_ref[...]
    hn = _consume(agg_ref, xsp_ref, dinvb, bgb[...], bng[...], bnb[...])
    m = 0.5 * (hn[0:N, :] + hn[N:NT, :])
    y_out[...] = _pool_y(P_ref[...], m, fWt[...], fb[...])
    xs = _conv_stage(hn, dinvb, W1t[...], b1[...], g1[...], be1[...], W2t[...], b2[...],
                     Wgt[...])
    _write_xs(xs_out, xs)


def _tc_final_body(agg_ref, xsp_ref, dinvb_ref, P_ref, bgb, bng, bnb, fWt, fb, y_out):
    hn = _consume(agg_ref, xsp_ref, dinvb_ref[...], bgb[...], bng[...], bnb[...])
    m = 0.5 * (hn[0:N, :] + hn[N:NT, :])
    y_out[...] = _pool_y(P_ref[...], m, fWt[...], fb[...])


_XS_TYPE = jax.ShapeDtypeStruct((XT, D), jnp.float32)
_Y_TYPE = jax.ShapeDtypeStruct((G, OUT), jnp.float32)

_tc0 = pl.pallas_call(_tc0_body, out_shape=[_XS_TYPE, _Y_TYPE])
_tc_mid = pl.pallas_call(_tc_mid_body, out_shape=[_XS_TYPE, _Y_TYPE])
_tc_final = pl.pallas_call(_tc_final_body, out_shape=_Y_TYPE)


def kernel(x, edge_index, batch, params):
    f32 = jnp.float32
    drop = jax.random.bernoulli(jax.random.key(42), 0.2, (R, N))
    scale = 1.0 - drop.astype(f32)
    xf0 = jnp.concatenate([x * scale[0][:, None], x * scale[1][:, None]], axis=0)
    xm0 = x * ((scale[0] + scale[1]) * 0.5)[:, None]

    em = jnp.max(edge_index) + 1
    row0 = edge_index[0].reshape(NS, E // NS)
    col0 = edge_index[1].reshape(NS, E // NS)
    padr = jnp.zeros((NS, EPAD), jnp.int32)
    padc = N + (jnp.broadcast_to(jnp.arange(EPAD, dtype=jnp.int32) % NTRASH, (NS, EPAD)))
    colp = jnp.concatenate([col0, padc], axis=1)
    rows0 = jnp.concatenate([row0, padr], axis=1)
    rows_c = jnp.stack([rows0, rows0 + em])
    cols_c = jnp.broadcast_to(colp[None], (NCORES, NS, EPT))
    idx_mp = jnp.stack(
        [rows_c.reshape(NCORES, NS, PAIRS, SBS, CHUNK),
         cols_c.reshape(NCORES, NS, PAIRS, SBS, CHUNK)], axis=4)
    zinit = jnp.zeros((ACC_ROWS, D), f32)

    mp_kernel, deg_kernel = _sc_kernels()

    ztrash = jnp.zeros((NTRASH, D), f32)
    ztail = jnp.zeros((CANVAS - ACC_ROWS, D), f32)

    def recombine(out_cat):
        out1 = jnp.concatenate([out_cat[ACC_ROWS : ACC_ROWS + N], ztrash], axis=0)
        canvas = jnp.concatenate([out_cat[0:ACC_ROWS], ztail], axis=0)
        return lax.dynamic_update_slice(canvas, out1, (em, 0))

    deg_cat = deg_kernel(idx_mp, zinit, jnp.ones((CHUNK, D), f32))
    deg = recombine(deg_cat)[0:NT, 0] + 1.0
    dinv = jnp.where(deg > 0, lax.rsqrt(deg), 0.0)
    dinvb = jnp.broadcast_to(dinv[:, None], (NT, D))

    P = (batch[:, None] == jnp.arange(G, dtype=batch.dtype)[None, :]).astype(f32)

    def r2(v):
        return v.reshape(1, -1)

    convs = params["convs"]
    fcs = params["fcs"]
    bns = params["bns"]

    c0 = convs[0]
    xs, y = _tc0(
        xf0, xm0, dinvb, P,
        c0["W1"].T, r2(c0["b1"]), r2(c0["g1"]), r2(c0["be1"]),
        c0["W2"].T, r2(c0["b2"]), c0["Wg"].T,
        fcs[0]["W"].T, r2(fcs[0]["b"]),
    )
    out = y
    for i in range(NUM_LAYERS):
        agg = recombine(mp_kernel(xs, idx_mp, zinit))
        bgb = r2(convs[i]["bg"])
        bng, bnb = r2(bns[i]["g"]), r2(bns[i]["b"])
        fWt, fb = fcs[i + 1]["W"].T, r2(fcs[i + 1]["b"])
        if i < NUM_LAYERS - 1:
            cn = convs[i + 1]
            xs_next, y = _tc_mid(
                agg, xs, dinvb, P, bgb, bng, bnb,
                cn["W1"].T, r2(cn["b1"]), r2(cn["g1"]), r2(cn["be1"]),
                cn["W2"].T, r2(cn["b2"]), cn["Wg"].T,
                fWt, fb,
            )
            xs = xs_next
        else:
            y = _tc_final(agg, xs, dinvb, P, bgb, bng, bnb, fWt, fb)
        out = out + y
    return out
